# Initial kernel scaffold; baseline (speedup 1.0000x reference)
import jax
import jax.numpy as jnp
from jax.experimental import pallas as pl


def kernel(logits, targets):
    raise NotImplementedError("write your pallas kernel here")


# speedup vs baseline: 49.5958x; 49.5958x over previous
import functools

import jax
import jax.numpy as jnp
import numpy as np
from jax import lax
from jax.experimental import pallas as pl
from jax.experimental.pallas import tpu as pltpu
from jax.experimental.pallas import tpu_sc as plsc

ALPHA = 0.5
NUM_THETA = 180
RHO_BINS = 512
LINE_THRESH = 50.0
H = W = 512

_DIAG = float(np.sqrt(2.0) * 512.0)
_K = (RHO_BINS - 1) / (2.0 * _DIAG)

_thetas = np.linspace(-np.pi / 2.0, np.pi / 2.0, NUM_THETA).astype(np.float32)
_A = (np.cos(_thetas.astype(np.float64)) * _K).astype(np.float32)
_B = (np.sin(_thetas.astype(np.float64)) * _K).astype(np.float32)
_C = np.float32(_DIAG * _K)
_ATAB = np.repeat(_A[:, None], 16, axis=1).reshape(-1)
_BTAB = np.repeat(_B[:, None], 16, axis=1).reshape(-1)
_XVF = np.arange(W, dtype=np.float32)

_NC, _NS = 2, 16
_NW = _NC * _NS
_TPW = NUM_THETA * 8 // _NW
_G = 9
_NGRP = _TPW // _G
_CH_ROWS = 32
_NCH = H // _CH_ROWS
_PIX = _CH_ROWS * W
_LANE_STRIDE = RHO_BINS
_THETA_STRIDE = 16 * RHO_BINS


def _hough_sc_body(masks_hbm, atab_hbm, btab_hbm, xvf_hbm, out_hbm,
                   accv, chunkv, atabv, btabv, xvfv, histv):
    wid = lax.axis_index("s") * _NC + lax.axis_index("c")
    m = wid % 8
    tbase = (wid // 8) * _TPW

    pltpu.sync_copy(atab_hbm, atabv)
    pltpu.sync_copy(btab_hbm, btabv)
    pltpu.sync_copy(xvf_hbm, xvfv)

    lane = lax.iota(jnp.int32, 16)
    zero16 = jnp.zeros((16,), jnp.float32)
    cvec = jnp.full((16,), _C, jnp.float32)

    def group_body(g, _):
        def zbody(i, _):
            accv[pl.ds(i * 16, 16)] = zero16
            return 0
        lax.fori_loop(0, _G * _THETA_STRIDE // 16, zbody, 0)

        def chunk_body(ci, _):
            pltpu.sync_copy(masks_hbm.at[m, pl.ds(ci * _PIX, _PIX)], chunkv)
            ybase = ci * _CH_ROWS

            def theta_body(tl, _):
                t = tbase + g * _G + tl
                av = atabv[pl.ds(t * 16, 16)]
                bv = btabv[pl.ds(t * 16, 16)]
                offv = lane * _LANE_STRIDE + tl * _THETA_STRIDE

                def row_body(r, _):
                    yv = jnp.full((16,), ybase + r, jnp.int32).astype(jnp.float32)
                    cyv = yv * bv + cvec

                    def xb_body(xb, _):
                        xv = xvfv[pl.ds(xb * 16, 16)]
                        w = chunkv[pl.ds(r * W + xb * 16, 16)]
                        idx = (xv * av + cyv).astype(jnp.int32) + offv
                        plsc.addupdate_scatter(accv, [idx], w)
                        return 0
                    lax.fori_loop(0, W // 16, xb_body, 0)
                    return 0
                lax.fori_loop(0, _CH_ROWS, row_body, 0)
                return 0
            lax.fori_loop(0, _G, theta_body, 0)
            return 0
        lax.fori_loop(0, _NCH, chunk_body, 0)

        def out_body(tl, _):
            def red_body(c, _):
                base = tl * _THETA_STRIDE + c * 16
                s = accv[pl.ds(base, 16)]
                for l in range(1, 16):
                    s = s + accv[pl.ds(base + l * _LANE_STRIDE, 16)]
                histv[pl.ds(c * 16, 16)] = s
                return 0
            lax.fori_loop(0, RHO_BINS // 16, red_body, 0)
            t = tbase + g * _G + tl
            pltpu.sync_copy(histv, out_hbm.at[m, t])
            return 0
        lax.fori_loop(0, _G, out_body, 0)
        return 0
    lax.fori_loop(0, _NGRP, group_body, 0)


@functools.cache
def _hough_sc():
    return pl.kernel(
        _hough_sc_body,
        out_type=jax.ShapeDtypeStruct((8, NUM_THETA, RHO_BINS), jnp.float32),
        mesh=plsc.VectorSubcoreMesh(core_axis_name="c", subcore_axis_name="s",
                                    num_cores=_NC, num_subcores=_NS),
        compiler_params=pltpu.CompilerParams(needs_layout_passes=False),
        scratch_types=[
            pltpu.VMEM((_G * _THETA_STRIDE,), jnp.float32),
            pltpu.VMEM((_PIX,), jnp.float32),
            pltpu.VMEM((NUM_THETA * 16,), jnp.float32),
            pltpu.VMEM((NUM_THETA * 16,), jnp.float32),
            pltpu.VMEM((W,), jnp.float32),
            pltpu.VMEM((RHO_BINS,), jnp.float32),
        ],
    )


def _prep_body(lg_ref, tg_ref, mask_ref, sums_ref):
    i = pl.program_id(0)
    lg = lg_ref[0]
    tg = tg_ref[0]
    probs = jax.nn.sigmoid(lg)
    is_pred = (i < 4)
    mask = jnp.where(is_pred, (lg > 0.0).astype(jnp.float32),
                     (tg > 0.5).astype(jnp.float32))
    mask_ref[0] = mask
    pf = is_pred.astype(jnp.float32)
    s0 = jnp.sum(probs * tg) * pf
    s1 = jnp.sum(probs) * pf
    s2 = jnp.sum(tg) * (1.0 - pf)
    lanes = lax.broadcasted_iota(jnp.int32, (1, 1, 128), 2)
    sums_ref[...] = jnp.where(
        lanes == 0, s0, jnp.where(lanes == 1, s1, jnp.where(lanes == 2, s2, 0.0)))


def _post_body(ap_ref, at_ref, sums_ref):
    ap = ap_ref[0]
    at = at_ref[0]
    tp = jnp.where(ap >= LINE_THRESH, ap, 0.0)
    tt = jnp.where(at >= LINE_THRESH, at, 0.0)
    php = tp / jnp.maximum(jnp.max(tp), 1e-12)
    pht = tt / jnp.maximum(jnp.max(tt), 1e-12)
    s0 = jnp.sum(php * pht)
    s1 = jnp.sum(php)
    s2 = jnp.sum(pht)
    lanes = lax.broadcasted_iota(jnp.int32, (1, 1, 128), 2)
    sums_ref[...] = jnp.where(
        lanes == 0, s0, jnp.where(lanes == 1, s1, jnp.where(lanes == 2, s2, 0.0)))


def kernel(logits, targets):
    lg = logits.reshape(4, H, W)
    tg = targets.reshape(4, H, W)

    masks, sums1 = pl.pallas_call(
        _prep_body,
        grid=(8,),
        in_specs=[
            pl.BlockSpec((1, H, W), lambda i: (i % 4, 0, 0)),
            pl.BlockSpec((1, H, W), lambda i: (i % 4, 0, 0)),
        ],
        out_specs=[
            pl.BlockSpec((1, H, W), lambda i: (i, 0, 0)),
            pl.BlockSpec((1, 1, 128), lambda i: (i, 0, 0)),
        ],
        out_shape=[
            jax.ShapeDtypeStruct((8, H, W), jnp.float32),
            jax.ShapeDtypeStruct((8, 1, 128), jnp.float32),
        ],
    )(lg, tg)

    acc8 = _hough_sc()(masks.reshape(8, H * W),
                       jnp.asarray(_ATAB), jnp.asarray(_BTAB), jnp.asarray(_XVF))

    sums3 = pl.pallas_call(
        _post_body,
        grid=(4,),
        in_specs=[
            pl.BlockSpec((1, NUM_THETA, RHO_BINS), lambda i: (i, 0, 0)),
            pl.BlockSpec((1, NUM_THETA, RHO_BINS), lambda i: (i + 4, 0, 0)),
        ],
        out_specs=pl.BlockSpec((1, 1, 128), lambda i: (i, 0, 0)),
        out_shape=jax.ShapeDtypeStruct((4, 1, 128), jnp.float32),
    )(acc8, acc8)

    i1 = jnp.sum(sums1[:, 0, 0])
    card1 = jnp.sum(sums1[:, 0, 1]) + jnp.sum(sums1[:, 0, 2])
    loss_img = 1.0 - 2.0 * i1 / jnp.maximum(card1, 1e-7)

    i2 = jnp.sum(sums3[:, 0, 0])
    card2 = jnp.sum(sums3[:, 0, 1]) + jnp.sum(sums3[:, 0, 2])
    loss_h = 1.0 - 2.0 * i2 / jnp.maximum(card2, 1e-7)

    return ((1.0 - ALPHA) * loss_img + ALPHA * loss_h).astype(jnp.float32)


# speedup vs baseline: 176.5587x; 3.5599x over previous
import functools

import jax
import jax.numpy as jnp
import numpy as np
from jax import lax
from jax.experimental import pallas as pl
from jax.experimental.pallas import tpu as pltpu
from jax.experimental.pallas import tpu_sc as plsc

ALPHA = 0.5
NUM_THETA = 180
RHO_BINS = 512
LINE_THRESH = 50.0
H = W = 512

_DIAG = float(np.sqrt(2.0) * 512.0)
_K = (RHO_BINS - 1) / (2.0 * _DIAG)

_thetas = np.linspace(-np.pi / 2.0, np.pi / 2.0, NUM_THETA).astype(np.float32)
_A = (np.cos(_thetas.astype(np.float64)) * _K).astype(np.float32)
_B = (np.sin(_thetas.astype(np.float64)) * _K).astype(np.float32)
_C = np.float32(_DIAG * _K)
_ATAB = np.repeat(_A[:, None], 16, axis=1).reshape(-1)
_BTAB = np.repeat(_B[:, None], 16, axis=1).reshape(-1)
_XVF = np.arange(W, dtype=np.float32)

_NC, _NS = 2, 16
_NW = _NC * _NS
_TPW = NUM_THETA * 8 // _NW
_G = 9
_NGRP = _TPW // _G
_CH_ROWS = 32
_NCH = H // _CH_ROWS
_PIX = _CH_ROWS * W
_LANE_STRIDE = RHO_BINS + 1
_THETA_STRIDE = 16 * _LANE_STRIDE


def _hough_sc_body(masks_hbm, atab_hbm, btab_hbm, xvf_hbm, out_hbm,
                   accv, chunkv, atabv, btabv, xvfv, histv):
    wid = lax.axis_index("s") * _NC + lax.axis_index("c")
    m = wid % 8
    tbase = (wid // 8) * _TPW

    pltpu.sync_copy(atab_hbm, atabv)
    pltpu.sync_copy(btab_hbm, btabv)
    pltpu.sync_copy(xvf_hbm, xvfv)

    lane = lax.iota(jnp.int32, 16)
    zero16 = jnp.zeros((16,), jnp.float32)
    cvec = jnp.full((16,), _C, jnp.float32)

    def group_body(g, _):
        def zbody(i, _):
            accv[pl.ds(i * 16, 16)] = zero16
            return 0
        lax.fori_loop(0, _G * _THETA_STRIDE // 16, zbody, 0)

        def chunk_body(ci, _):
            pltpu.sync_copy(masks_hbm.at[m, pl.ds(ci * _PIX, _PIX)], chunkv)
            ybase = ci * _CH_ROWS

            def theta_body(tl, _):
                t = tbase + g * _G + tl
                av = atabv[pl.ds(t * 16, 16)]
                bv = btabv[pl.ds(t * 16, 16)]
                dav = av * 16.0
                offv = lane * _LANE_STRIDE + tl * _THETA_STRIDE
                base0 = xvfv[pl.ds(0, 16)] * av

                def row_body(r, _):
                    yv = jnp.full((16,), ybase + r, jnp.int32).astype(jnp.float32)
                    idxf = base0 + (yv * bv + cvec)
                    rbase = r * W
                    for xb in range(W // 16):
                        w = chunkv[pl.ds(rbase + xb * 16, 16)]
                        idx = idxf.astype(jnp.int32) + offv
                        plsc.addupdate_scatter(accv, [idx], w)
                        idxf = idxf + dav
                    return 0
                lax.fori_loop(0, _CH_ROWS, row_body, 0)
                return 0
            lax.fori_loop(0, _G, theta_body, 0)
            return 0
        lax.fori_loop(0, _NCH, chunk_body, 0)

        def out_body(tl, _):
            def red_body(c, _):
                base = tl * _THETA_STRIDE + c * 16
                s = accv[pl.ds(base, 16)]
                for l in range(1, 16):
                    s = s + accv[pl.ds(base + l * _LANE_STRIDE, 16)]
                histv[pl.ds(c * 16, 16)] = s
                return 0
            lax.fori_loop(0, RHO_BINS // 16, red_body, 0)
            t = tbase + g * _G + tl
            pltpu.sync_copy(histv, out_hbm.at[m, t])
            return 0
        lax.fori_loop(0, _G, out_body, 0)
        return 0
    lax.fori_loop(0, _NGRP, group_body, 0)


@functools.cache
def _hough_sc():
    return pl.kernel(
        _hough_sc_body,
        out_type=jax.ShapeDtypeStruct((8, NUM_THETA, RHO_BINS), jnp.float32),
        mesh=plsc.VectorSubcoreMesh(core_axis_name="c", subcore_axis_name="s",
                                    num_cores=_NC, num_subcores=_NS),
        compiler_params=pltpu.CompilerParams(needs_layout_passes=False),
        scratch_types=[
            pltpu.VMEM((_G * _THETA_STRIDE,), jnp.float32),
            pltpu.VMEM((_PIX,), jnp.float32),
            pltpu.VMEM((NUM_THETA * 16,), jnp.float32),
            pltpu.VMEM((NUM_THETA * 16,), jnp.float32),
            pltpu.VMEM((W,), jnp.float32),
            pltpu.VMEM((RHO_BINS,), jnp.float32),
        ],
    )


def _prep_body(lg_ref, tg_ref, mask_ref, sums_ref):
    i = pl.program_id(0)
    lg = lg_ref[0]
    tg = tg_ref[0]
    probs = jax.nn.sigmoid(lg)
    is_pred = (i < 4)
    mask = jnp.where(is_pred, (lg > 0.0).astype(jnp.float32),
                     (tg > 0.5).astype(jnp.float32))
    mask_ref[0] = mask
    pf = is_pred.astype(jnp.float32)
    s0 = jnp.sum(probs * tg) * pf
    s1 = jnp.sum(probs) * pf
    s2 = jnp.sum(tg) * (1.0 - pf)
    lanes = lax.broadcasted_iota(jnp.int32, (1, 1, 128), 2)
    sums_ref[...] = jnp.where(
        lanes == 0, s0, jnp.where(lanes == 1, s1, jnp.where(lanes == 2, s2, 0.0)))


def _post_body(ap_ref, at_ref, sums_ref):
    ap = ap_ref[0]
    at = at_ref[0]
    tp = jnp.where(ap >= LINE_THRESH, ap, 0.0)
    tt = jnp.where(at >= LINE_THRESH, at, 0.0)
    php = tp / jnp.maximum(jnp.max(tp), 1e-12)
    pht = tt / jnp.maximum(jnp.max(tt), 1e-12)
    s0 = jnp.sum(php * pht)
    s1 = jnp.sum(php)
    s2 = jnp.sum(pht)
    lanes = lax.broadcasted_iota(jnp.int32, (1, 1, 128), 2)
    sums_ref[...] = jnp.where(
        lanes == 0, s0, jnp.where(lanes == 1, s1, jnp.where(lanes == 2, s2, 0.0)))


def kernel(logits, targets):
    lg = logits.reshape(4, H, W)
    tg = targets.reshape(4, H, W)

    masks, sums1 = pl.pallas_call(
        _prep_body,
        grid=(8,),
        in_specs=[
            pl.BlockSpec((1, H, W), lambda i: (i % 4, 0, 0)),
            pl.BlockSpec((1, H, W), lambda i: (i % 4, 0, 0)),
        ],
        out_specs=[
            pl.BlockSpec((1, H, W), lambda i: (i, 0, 0)),
            pl.BlockSpec((1, 1, 128), lambda i: (i, 0, 0)),
        ],
        out_shape=[
            jax.ShapeDtypeStruct((8, H, W), jnp.float32),
            jax.ShapeDtypeStruct((8, 1, 128), jnp.float32),
        ],
    )(lg, tg)

    acc8 = _hough_sc()(masks.reshape(8, H * W),
                       jnp.asarray(_ATAB), jnp.asarray(_BTAB), jnp.asarray(_XVF))

    sums3 = pl.pallas_call(
        _post_body,
        grid=(4,),
        in_specs=[
            pl.BlockSpec((1, NUM_THETA, RHO_BINS), lambda i: (i, 0, 0)),
            pl.BlockSpec((1, NUM_THETA, RHO_BINS), lambda i: (i + 4, 0, 0)),
        ],
        out_specs=pl.BlockSpec((1, 1, 128), lambda i: (i, 0, 0)),
        out_shape=jax.ShapeDtypeStruct((4, 1, 128), jnp.float32),
    )(acc8, acc8)

    i1 = jnp.sum(sums1[:, 0, 0])
    card1 = jnp.sum(sums1[:, 0, 1]) + jnp.sum(sums1[:, 0, 2])
    loss_img = 1.0 - 2.0 * i1 / jnp.maximum(card1, 1e-7)

    i2 = jnp.sum(sums3[:, 0, 0])
    card2 = jnp.sum(sums3[:, 0, 1]) + jnp.sum(sums3[:, 0, 2])
    loss_h = 1.0 - 2.0 * i2 / jnp.maximum(card2, 1e-7)

    return ((1.0 - ALPHA) * loss_img + ALPHA * loss_h).astype(jnp.float32)
